# Initial kernel scaffold; baseline (speedup 1.0000x reference)
import jax
import jax.numpy as jnp
from jax.experimental import pallas as pl


def kernel(xyz, new_xyz, features):
    raise NotImplementedError("write your pallas kernel here")


# speedup vs baseline: 13.4474x; 13.4474x over previous
import functools
import numpy as np
import jax
import jax.numpy as jnp
from jax import lax
from jax.experimental import pallas as pl
from jax.experimental.pallas import tpu as pltpu
from jax.experimental.pallas import tpu_sc as plsc

B, N, M, C, S = 4, 8192, 1024, 64, 32
N1 = N + 1
D = 128
R2 = np.float32(0.2 * 0.2)

BM = B * M
NW = 32
RPW = BM * S // NW
CHUNK = 512
NCH = RPW // CHUNK

M_BLK = 512
N_BLK = 4096
M_SEL = 256
M_BLK2 = 256


def _d2_body(nxyz_ref, xyz_ref, d2_ref):
    cxyz = nxyz_ref[0]
    pxyz = xyz_ref[0]
    cb = cxyz.astype(jnp.bfloat16)
    pb = pxyz.astype(jnp.bfloat16)
    dot = lax.dot_general(cb, pb, (((1,), (1,)), ((), ())),
                          preferred_element_type=jnp.float32)
    cx, cy, cz = cxyz[:, 0], cxyz[:, 1], cxyz[:, 2]
    px, py, pz = pxyz[:, 0], pxyz[:, 1], pxyz[:, 2]
    nx2 = cx * cx + cy * cy + cz * cz
    x2 = px * px + py * py + pz * pz
    d2_ref[0] = (nx2[:, None] + x2[None, :]) - 2.0 * dot


def _compute_d2(new_xyz, xyz):
    grid = (B, M // M_BLK, N // N_BLK)
    return pl.pallas_call(
        _d2_body,
        grid=grid,
        in_specs=[
            pl.BlockSpec((1, M_BLK, 3), lambda b, mc, nc: (b, mc, 0)),
            pl.BlockSpec((1, N_BLK, 3), lambda b, mc, nc: (b, nc, 0)),
        ],
        out_specs=pl.BlockSpec((1, M_BLK, N_BLK), lambda b, mc, nc: (b, mc, nc)),
        out_shape=jax.ShapeDtypeStruct((B, M, N), jnp.float32),
    )(new_xyz, xyz)


def _sel_body(d2_ref, idx_ref):
    d2 = d2_ref[0]
    j = lax.broadcasted_iota(jnp.int32, (M_SEL, N), 1) + 1
    keys = jnp.where(d2 < R2, j, jnp.int32(N1))
    cols = []
    for s in range(S):
        cur = jnp.min(keys, axis=1)
        cols.append(cur)
        keys = jnp.where(keys == cur[:, None], jnp.int32(N1), keys)
    sel = jnp.stack(cols, axis=1)
    validm = sel < N1
    first = sel[:, 0:1]
    first = jnp.where(first < N1, first, 0)
    idx_ref[0] = jnp.where(validm, sel, first)


def _select(d2):
    grid = (B, M // M_SEL)
    return pl.pallas_call(
        _sel_body,
        grid=grid,
        in_specs=[pl.BlockSpec((1, M_SEL, N), lambda b, mc: (b, mc, 0))],
        out_specs=pl.BlockSpec((1, M_SEL, S), lambda b, mc: (b, mc, 0)),
        out_shape=jax.ShapeDtypeStruct((B, M, S), jnp.int32),
    )(d2)


def _gather_body(table_hbm, idx_hbm, out_hbm, idx_v, rows_v, sem):
    nc = 2
    wid = lax.axis_index("s") * nc + lax.axis_index("c")
    base = wid * RPW
    for ch in range(NCH):
        off = base + ch * CHUNK
        pltpu.sync_copy(idx_hbm.at[pl.ds(off, CHUNK)], idx_v)
        pltpu.async_copy(table_hbm.at[idx_v], rows_v, sem).wait()
        pltpu.sync_copy(rows_v, out_hbm.at[pl.ds(off, CHUNK)])


def _sc_gather(table, idx_flat):
    mesh = plsc.VectorSubcoreMesh(core_axis_name="c", subcore_axis_name="s")
    kern = functools.partial(
        pl.kernel,
        mesh=mesh,
        out_type=jax.ShapeDtypeStruct((BM * S, D), jnp.float32),
        scratch_types=[
            pltpu.VMEM((CHUNK,), jnp.int32),
            pltpu.VMEM((CHUNK, D), jnp.float32),
            pltpu.SemaphoreType.DMA,
        ],
    )(_gather_body)
    return kern(table, idx_flat)


def _asm_body(idx_ref, g_ref, nxyz_ref, out_ref, valid_ref):
    idxb = idx_ref[0]
    gt = g_ref[0]
    ft = gt[:, :, 64:]
    ftt = jnp.transpose(ft.reshape(M_BLK2 * S, C))
    ftt = ftt.reshape(C, M_BLK2, S)
    xyzs = gt[:, :, 0:3]
    xg = jnp.transpose(xyzs.reshape(M_BLK2 * S, 3)).reshape(3, M_BLK2, S)
    ctr = jnp.transpose(nxyz_ref[0])[:, :, None]
    g = xg - ctr
    g = jnp.where(idxb[None] == 0, jnp.zeros_like(g), g)
    out_ref[0] = jnp.concatenate([g, ftt], axis=0)
    valid_ref[0] = (jnp.sum(idxb, axis=-1) != 0).astype(jnp.int32)[:, None]


def _assemble(idx, gath, new_xyz):
    grid = (B, M // M_BLK2)
    out, valid = pl.pallas_call(
        _asm_body,
        grid=grid,
        in_specs=[
            pl.BlockSpec((1, M_BLK2, S), lambda b, mc: (b, mc, 0)),
            pl.BlockSpec((1, M_BLK2, S, D), lambda b, mc: (b, mc, 0, 0)),
            pl.BlockSpec((1, M_BLK2, 3), lambda b, mc: (b, mc, 0)),
        ],
        out_specs=[
            pl.BlockSpec((1, 3 + C, M_BLK2, S), lambda b, mc: (b, 0, mc, 0)),
            pl.BlockSpec((1, M_BLK2, 1), lambda b, mc: (b, mc, 0)),
        ],
        out_shape=[
            jax.ShapeDtypeStruct((B, 3 + C, M, S), jnp.float32),
            jax.ShapeDtypeStruct((B, M, 1), jnp.int32),
        ],
    )(idx, gath, new_xyz)
    return out, valid


def kernel(xyz, new_xyz, features):
    d2 = _compute_d2(new_xyz, xyz)
    idx = _select(d2)

    xyz_p = jnp.concatenate(
        [jnp.full((B, 1, 3), 1000.0, jnp.float32), xyz], axis=1)
    featT = jnp.concatenate(
        [jnp.zeros((B, 1, C), jnp.float32),
         jnp.transpose(features, (0, 2, 1))], axis=1)
    table = jnp.concatenate(
        [xyz_p, jnp.zeros((B, N1, 61), jnp.float32), featT],
        axis=2).reshape(B * N1, D)

    boff = (jnp.arange(B, dtype=jnp.int32) * N1)[:, None, None]
    idx_flat = (idx + boff).reshape(BM * S)

    gath = _sc_gather(table, idx_flat).reshape(B, M, S, D)
    new_features, valid = _assemble(idx, gath, new_xyz)
    return (new_features, valid[..., 0].astype(jnp.bool_))
